# baseline (speedup 1.0000x reference)
import functools

import jax
import jax.numpy as jnp
from jax.experimental import pallas as pl

ALPHA = 3.0
K = 8
TR = 256


def _nodevec_body(x_ref, w1_ref, b1_ref, w2_ref, b2_ref, nv1_ref, nv2_ref):
    x = x_ref[0]
    nv1_ref[0] = jnp.tanh(ALPHA * (jnp.dot(x, w1_ref[...],
                                           preferred_element_type=jnp.float32)
                                   + b1_ref[...]))
    nv2_ref[0] = jnp.tanh(ALPHA * (jnp.dot(x, w2_ref[...],
                                           preferred_element_type=jnp.float32)
                                   + b2_ref[...]))


def _adj_topk_body(nv1_ref, nv2_ref, out_ref, *, n_rows):
    r = pl.program_id(1)
    n1 = nv1_ref[0]
    n2 = nv2_ref[0]
    rows1 = nv1_ref[0, pl.ds(r * n_rows, n_rows), :]
    rows2 = nv2_ref[0, pl.ds(r * n_rows, n_rows), :]

    contract = (((1,), (1,)), ((), ()))
    s1 = jax.lax.dot_general(rows1, n2, contract,
                             preferred_element_type=jnp.float32)
    s2 = jax.lax.dot_general(rows2, n1, contract,
                             preferred_element_type=jnp.float32)
    adj = jnp.maximum(jnp.tanh(ALPHA * (s1 - s2)), 0.0)

    idx = jax.lax.broadcasted_iota(jnp.int32, adj.shape, 1)
    work = adj
    n_cols = adj.shape[1]
    for _ in range(K):
        m = jnp.max(work, axis=1, keepdims=True)
        mi = jnp.min(jnp.where(work == m, idx, n_cols), axis=1, keepdims=True)
        work = jnp.where(idx == mi, -1.0, work)

    out_ref[0] = jnp.where(work < 0.0, adj, 0.0)


def kernel(X, W1, b1, W2, b2):
    B, N, FD = X.shape
    D = W1.shape[1]

    nv1, nv2 = pl.pallas_call(
        _nodevec_body,
        grid=(B,),
        in_specs=[
            pl.BlockSpec((1, N, FD), lambda b: (b, 0, 0)),
            pl.BlockSpec((FD, D), lambda b: (0, 0)),
            pl.BlockSpec((D,), lambda b: (0,)),
            pl.BlockSpec((FD, D), lambda b: (0, 0)),
            pl.BlockSpec((D,), lambda b: (0,)),
        ],
        out_specs=[
            pl.BlockSpec((1, N, D), lambda b: (b, 0, 0)),
            pl.BlockSpec((1, N, D), lambda b: (b, 0, 0)),
        ],
        out_shape=[
            jax.ShapeDtypeStruct((B, N, D), jnp.float32),
            jax.ShapeDtypeStruct((B, N, D), jnp.float32),
        ],
    )(X, W1, b1, W2, b2)

    out = pl.pallas_call(
        functools.partial(_adj_topk_body, n_rows=TR),
        grid=(B, N // TR),
        in_specs=[
            pl.BlockSpec((1, N, D), lambda b, r: (b, 0, 0)),
            pl.BlockSpec((1, N, D), lambda b, r: (b, 0, 0)),
        ],
        out_specs=pl.BlockSpec((1, TR, N), lambda b, r: (b, r, 0)),
        out_shape=jax.ShapeDtypeStruct((B, N, N), jnp.float32),
    )(nv1, nv2)
    return out


# speedup vs baseline: 1.2030x; 1.2030x over previous
import functools

import jax
import jax.numpy as jnp
from jax.experimental import pallas as pl

ALPHA = 3.0
K = 8
TR = 256


def _nodevec_body(x_ref, w1_ref, b1_ref, w2_ref, b2_ref, nv1_ref, nv2_ref):
    x = x_ref[0]
    nv1_ref[0] = jnp.tanh(ALPHA * (jnp.dot(x, w1_ref[...],
                                           preferred_element_type=jnp.float32)
                                   + b1_ref[...]))
    nv2_ref[0] = jnp.tanh(ALPHA * (jnp.dot(x, w2_ref[...],
                                           preferred_element_type=jnp.float32)
                                   + b2_ref[...]))


def _adj_topk_body(nv1_ref, nv2_ref, out_ref, *, n_rows):
    r = pl.program_id(1)
    n1 = nv1_ref[0]
    n2 = nv2_ref[0]
    rows1 = nv1_ref[0, pl.ds(r * n_rows, n_rows), :]
    rows2 = nv2_ref[0, pl.ds(r * n_rows, n_rows), :]

    contract = (((1,), (1,)), ((), ()))
    s1 = jax.lax.dot_general(rows1, n2, contract,
                             preferred_element_type=jnp.float32)
    s2 = jax.lax.dot_general(rows2, n1, contract,
                             preferred_element_type=jnp.float32)
    adj = jnp.maximum(jnp.tanh(ALPHA * (s1 - s2)), 0.0)

    idx = jax.lax.broadcasted_iota(jnp.int32, adj.shape, 1).astype(jnp.float32)
    work = adj
    n_cols = float(adj.shape[1])
    for _ in range(K):
        m = jnp.max(work, axis=1, keepdims=True)
        mi = jnp.min(jnp.where(work == m, idx, n_cols), axis=1, keepdims=True)
        work = jnp.where(idx == mi, -1.0, work)

    out_ref[0] = jnp.where(work < 0.0, adj, 0.0)


def kernel(X, W1, b1, W2, b2):
    B, N, FD = X.shape
    D = W1.shape[1]

    nv1, nv2 = pl.pallas_call(
        _nodevec_body,
        grid=(B,),
        in_specs=[
            pl.BlockSpec((1, N, FD), lambda b: (b, 0, 0)),
            pl.BlockSpec((FD, D), lambda b: (0, 0)),
            pl.BlockSpec((D,), lambda b: (0,)),
            pl.BlockSpec((FD, D), lambda b: (0, 0)),
            pl.BlockSpec((D,), lambda b: (0,)),
        ],
        out_specs=[
            pl.BlockSpec((1, N, D), lambda b: (b, 0, 0)),
            pl.BlockSpec((1, N, D), lambda b: (b, 0, 0)),
        ],
        out_shape=[
            jax.ShapeDtypeStruct((B, N, D), jnp.float32),
            jax.ShapeDtypeStruct((B, N, D), jnp.float32),
        ],
    )(X, W1, b1, W2, b2)

    out = pl.pallas_call(
        functools.partial(_adj_topk_body, n_rows=TR),
        grid=(B, N // TR),
        in_specs=[
            pl.BlockSpec((1, N, D), lambda b, r: (b, 0, 0)),
            pl.BlockSpec((1, N, D), lambda b, r: (b, 0, 0)),
        ],
        out_specs=pl.BlockSpec((1, TR, N), lambda b, r: (b, r, 0)),
        out_shape=jax.ShapeDtypeStruct((B, N, N), jnp.float32),
    )(nv1, nv2)
    return out


# speedup vs baseline: 1.2208x; 1.0148x over previous
import functools

import jax
import jax.numpy as jnp
from jax.experimental import pallas as pl

ALPHA = 3.0
K = 8
TR = 512


def _nodevec_body(x_ref, w1_ref, b1_ref, w2_ref, b2_ref, nv1_ref, nv2_ref):
    x = x_ref[0]
    nv1_ref[0] = jnp.tanh(ALPHA * (jnp.dot(x, w1_ref[...],
                                           preferred_element_type=jnp.float32)
                                   + b1_ref[...]))
    nv2_ref[0] = jnp.tanh(ALPHA * (jnp.dot(x, w2_ref[...],
                                           preferred_element_type=jnp.float32)
                                   + b2_ref[...]))


def _adj_topk_body(nv1_ref, nv2_ref, out_ref, *, n_rows):
    r = pl.program_id(1)
    n1 = nv1_ref[0]
    n2 = nv2_ref[0]
    rows1 = nv1_ref[0, pl.ds(r * n_rows, n_rows), :]
    rows2 = nv2_ref[0, pl.ds(r * n_rows, n_rows), :]

    contract = (((1,), (1,)), ((), ()))
    s1 = jax.lax.dot_general(rows1, n2, contract,
                             preferred_element_type=jnp.float32)
    s2 = jax.lax.dot_general(rows2, n1, contract,
                             preferred_element_type=jnp.float32)
    adj = jnp.maximum(jnp.tanh(ALPHA * (s1 - s2)), 0.0)

    idx = jax.lax.broadcasted_iota(jnp.int32, adj.shape, 1).astype(jnp.float32)
    work = adj
    n_cols = float(adj.shape[1])
    for _ in range(K):
        m = jnp.max(work, axis=1, keepdims=True)
        mi = jnp.min(jnp.where(work == m, idx, n_cols), axis=1, keepdims=True)
        work = jnp.where(idx == mi, -1.0, work)

    out_ref[0] = jnp.where(work < 0.0, adj, 0.0)


def kernel(X, W1, b1, W2, b2):
    B, N, FD = X.shape
    D = W1.shape[1]

    nv1, nv2 = pl.pallas_call(
        _nodevec_body,
        grid=(B,),
        in_specs=[
            pl.BlockSpec((1, N, FD), lambda b: (b, 0, 0)),
            pl.BlockSpec((FD, D), lambda b: (0, 0)),
            pl.BlockSpec((D,), lambda b: (0,)),
            pl.BlockSpec((FD, D), lambda b: (0, 0)),
            pl.BlockSpec((D,), lambda b: (0,)),
        ],
        out_specs=[
            pl.BlockSpec((1, N, D), lambda b: (b, 0, 0)),
            pl.BlockSpec((1, N, D), lambda b: (b, 0, 0)),
        ],
        out_shape=[
            jax.ShapeDtypeStruct((B, N, D), jnp.float32),
            jax.ShapeDtypeStruct((B, N, D), jnp.float32),
        ],
    )(X, W1, b1, W2, b2)

    out = pl.pallas_call(
        functools.partial(_adj_topk_body, n_rows=TR),
        grid=(B, N // TR),
        in_specs=[
            pl.BlockSpec((1, N, D), lambda b, r: (b, 0, 0)),
            pl.BlockSpec((1, N, D), lambda b, r: (b, 0, 0)),
        ],
        out_specs=pl.BlockSpec((1, TR, N), lambda b, r: (b, r, 0)),
        out_shape=jax.ShapeDtypeStruct((B, N, N), jnp.float32),
    )(nv1, nv2)
    return out


# speedup vs baseline: 1.2519x; 1.0255x over previous
import functools

import jax
import jax.numpy as jnp
from jax.experimental import pallas as pl
from jax.experimental.pallas import tpu as pltpu

ALPHA = 3.0
K = 8
TR = 512


def _nodevec_body(x_ref, w1_ref, b1_ref, w2_ref, b2_ref, nv1_ref, nv2_ref):
    x = x_ref[0]
    nv1_ref[0] = jnp.tanh(ALPHA * (jnp.dot(x, w1_ref[...],
                                           preferred_element_type=jnp.float32)
                                   + b1_ref[...]))
    nv2_ref[0] = jnp.tanh(ALPHA * (jnp.dot(x, w2_ref[...],
                                           preferred_element_type=jnp.float32)
                                   + b2_ref[...]))


def _adj_topk_body(nv1_ref, nv2_ref, out_ref, *, n_rows):
    r = pl.program_id(1)
    n1 = nv1_ref[0]
    n2 = nv2_ref[0]
    rows1 = nv1_ref[0, pl.ds(r * n_rows, n_rows), :]
    rows2 = nv2_ref[0, pl.ds(r * n_rows, n_rows), :]

    contract = (((1,), (1,)), ((), ()))
    s1 = jax.lax.dot_general(rows1, n2, contract,
                             preferred_element_type=jnp.float32)
    s2 = jax.lax.dot_general(rows2, n1, contract,
                             preferred_element_type=jnp.float32)
    adj = jnp.maximum(jnp.tanh(ALPHA * (s1 - s2)), 0.0)

    idx = jax.lax.broadcasted_iota(jnp.int32, adj.shape, 1).astype(jnp.float32)
    work = adj
    n_cols = float(adj.shape[1])
    for _ in range(K - 1):
        m = jnp.max(work, axis=1, keepdims=True)
        mi = jnp.min(jnp.where(work == m, idx, n_cols), axis=1, keepdims=True)
        work = jnp.where(idx == mi, -1.0, work)
    m = jnp.max(work, axis=1, keepdims=True)
    mi = jnp.min(jnp.where(work == m, idx, n_cols), axis=1, keepdims=True)

    out_ref[0] = jnp.where(jnp.logical_or(work < 0.0, idx == mi), adj, 0.0)


def kernel(X, W1, b1, W2, b2):
    B, N, FD = X.shape
    D = W1.shape[1]

    nv1, nv2 = pl.pallas_call(
        _nodevec_body,
        grid=(B,),
        in_specs=[
            pl.BlockSpec((1, N, FD), lambda b: (b, 0, 0)),
            pl.BlockSpec((FD, D), lambda b: (0, 0)),
            pl.BlockSpec((D,), lambda b: (0,)),
            pl.BlockSpec((FD, D), lambda b: (0, 0)),
            pl.BlockSpec((D,), lambda b: (0,)),
        ],
        out_specs=[
            pl.BlockSpec((1, N, D), lambda b: (b, 0, 0)),
            pl.BlockSpec((1, N, D), lambda b: (b, 0, 0)),
        ],
        out_shape=[
            jax.ShapeDtypeStruct((B, N, D), jnp.float32),
            jax.ShapeDtypeStruct((B, N, D), jnp.float32),
        ],
    )(X, W1, b1, W2, b2)

    out = pl.pallas_call(
        functools.partial(_adj_topk_body, n_rows=TR),
        grid=(B, N // TR),
        in_specs=[
            pl.BlockSpec((1, N, D), lambda b, r: (b, 0, 0)),
            pl.BlockSpec((1, N, D), lambda b, r: (b, 0, 0)),
        ],
        out_specs=pl.BlockSpec((1, TR, N), lambda b, r: (b, r, 0)),
        out_shape=jax.ShapeDtypeStruct((B, N, N), jnp.float32),
        compiler_params=pltpu.CompilerParams(
            dimension_semantics=("parallel", "arbitrary"),
        ),
    )(nv1, nv2)
    return out
